# baseline (speedup 1.0000x reference)
import functools

import jax
import jax.numpy as jnp
from jax.experimental import pallas as pl
from jax.experimental.pallas import tpu as pltpu

_ALPHA = 0.5


def _body(lt2_ref, labels_ref, emb_ref, out_ref, *, C, D):
    step = pl.program_id(0)

    A = emb_ref[...]
    lt2 = lt2_ref[...]
    lab = labels_ref[...]
    CD = C * D
    nrep = (CD + 4 * D - 1) // (4 * D)

    LT2 = jnp.tile(lt2, (1, 1, nrep))[:, :, :CD]
    h = A * (A - LT2)

    s = h
    for sh in (16, 8, 4, 2, 1):
        s = s + pltpu.roll(s, sh, 2)

    lsq = 0.25 * jnp.sum(lt2[:, :, :D] * lt2[:, :, :D], axis=2, keepdims=True)
    d = s + lsq

    m = jnp.max(lab, axis=2, keepdims=True)
    iota_c = jax.lax.broadcasted_iota(jnp.int32, lab.shape, 2)
    is_max = lab == m
    first_idx = jnp.min(jnp.where(is_max, iota_c, C), axis=2, keepdims=True)
    last_idx = jnp.max(jnp.where(is_max, iota_c, -1), axis=2, keepdims=True)

    p = jax.lax.broadcasted_iota(jnp.int32, A.shape, 2)
    jfull = jax.lax.shift_right_logical(p, 5)
    pmask = (p & (D - 1)) == D - 1

    d_correct = jnp.sum(
        jnp.where(pmask & (jfull == first_idx), d, 0.0), axis=(1, 2))
    hardest = jnp.min(
        jnp.where(pmask & (jfull != last_idx), d, jnp.float32(jnp.inf)),
        axis=(1, 2))
    loss = jnp.sum(jnp.maximum(d_correct - _ALPHA * hardest, 0.0))

    @pl.when(step == 0)
    def _init():
        out_ref[...] = jnp.zeros_like(out_ref)

    out_ref[...] = out_ref[...] + loss


def kernel(logits, labels, ans_emb, print_info):
    B, C = labels.shape
    D = logits.shape[1]
    nb = 4
    flat = ans_emb.reshape(B, 1, C * D)
    lt2 = jnp.tile(2.0 * logits, (1, 4)).reshape(B, 1, 4 * D)
    body = functools.partial(_body, C=C, D=D)
    out = pl.pallas_call(
        body,
        grid=(B // nb,),
        in_specs=[
            pl.BlockSpec((nb, 1, 4 * D), lambda i: (i, 0, 0)),
            pl.BlockSpec((nb, 1, C), lambda i: (i, 0, 0)),
            pl.BlockSpec((nb, 1, C * D), lambda i: (i, 0, 0)),
        ],
        out_specs=pl.BlockSpec((1, 1), lambda i: (0, 0)),
        out_shape=jax.ShapeDtypeStruct((1, 1), jnp.float32),
    )(lt2, labels.reshape(B, 1, C), flat)
    return out[0, 0]


# speedup vs baseline: 6.5018x; 6.5018x over previous
import functools

import jax
import jax.numpy as jnp
from jax.experimental import pallas as pl
from jax.experimental.pallas import tpu as pltpu

_ALPHA = 0.5
_R = 784


def _body(lt2_ref, labels_ref, emb_ref, out_ref, *, C, D):
    step = pl.program_id(0)

    A = emb_ref[...]
    lt2 = lt2_ref[...]
    lab = labels_ref[...]

    h = A * (A - lt2)
    gsel = (jax.lax.broadcasted_iota(jnp.int32, (128, 4), 0) // D
            == jax.lax.broadcasted_iota(jnp.int32, (128, 4), 1)
            ).astype(jnp.float32)
    s4 = jax.lax.dot_general(
        h, gsel,
        dimension_numbers=(((2,), (0,)), ((), ())),
        preferred_element_type=jnp.float32)
    lsq = jnp.sum(lt2 * lt2, axis=2, keepdims=True) / 16.0
    d = s4 + lsq

    m = jnp.max(lab, axis=2, keepdims=True)
    iota_c = jax.lax.broadcasted_iota(jnp.int32, lab.shape, 2)
    is_max = lab == m
    first_idx = jnp.min(jnp.where(is_max, iota_c, C), axis=2, keepdims=True)
    last_idx = jnp.max(jnp.where(is_max, iota_c, -1), axis=2, keepdims=True)

    j4 = (4 * jax.lax.broadcasted_iota(jnp.int32, d.shape, 1)
          + jax.lax.broadcasted_iota(jnp.int32, d.shape, 2))
    valid = j4 < C

    d_correct = jnp.sum(
        jnp.where(j4 == first_idx, d, 0.0), axis=(1, 2))
    hardest = jnp.min(
        jnp.where(valid & (j4 != last_idx), d, jnp.float32(jnp.inf)),
        axis=(1, 2))
    loss = jnp.sum(jnp.maximum(d_correct - _ALPHA * hardest, 0.0))

    @pl.when(step == 0)
    def _init():
        out_ref[...] = jnp.zeros_like(out_ref)

    out_ref[...] = out_ref[...] + loss


def kernel(logits, labels, ans_emb, print_info):
    B, C = labels.shape
    D = logits.shape[1]
    nb = 8
    flat = ans_emb.reshape(B, C * D)
    flat_p = jnp.pad(flat, ((0, 0), (0, _R * 128 - C * D))).reshape(
        B, _R, 128)
    lt2 = jnp.tile(2.0 * logits, (1, 4)).reshape(B, 1, 4 * D)
    body = functools.partial(_body, C=C, D=D)
    out = pl.pallas_call(
        body,
        grid=(B // nb,),
        in_specs=[
            pl.BlockSpec((nb, 1, 4 * D), lambda i: (i, 0, 0)),
            pl.BlockSpec((nb, 1, C), lambda i: (i, 0, 0)),
            pl.BlockSpec((nb, _R, 128), lambda i: (i, 0, 0)),
        ],
        out_specs=pl.BlockSpec((1, 1), lambda i: (0, 0)),
        out_shape=jax.ShapeDtypeStruct((1, 1), jnp.float32),
    )(lt2, labels.reshape(B, 1, C), flat_p)
    return out[0, 0]


# speedup vs baseline: 8.3423x; 1.2831x over previous
import functools

import jax
import jax.numpy as jnp
from jax.experimental import pallas as pl
from jax.experimental.pallas import tpu as pltpu

_ALPHA = 0.5
_R = 784


def _body(lt2_ref, labels_ref, emb_ref, out_ref, a_scr, *, C, D):
    step = pl.program_id(0)
    nb = lt2_ref.shape[0]
    nfull = (C * D) // 128

    for r in range(nfull):
        a_scr[:, pl.ds(r, 1), :] = emb_ref[:, pl.ds(r * 128, 128)].reshape(
            nb, 1, 128)
    tail = emb_ref[:, pl.ds(nfull * 128, C * D - nfull * 128)]
    a_scr[:, pl.ds(nfull, 1), :] = jnp.concatenate(
        [tail, jnp.zeros((nb, 96), jnp.float32)], axis=1).reshape(nb, 1, 128)

    @pl.when(step == 0)
    def _zero_tail():
        a_scr[:, pl.ds(nfull + 1, _R - nfull - 1), :] = jnp.zeros(
            (nb, _R - nfull - 1, 128), jnp.float32)

    A = a_scr[...]
    lt2 = lt2_ref[...]
    lab = labels_ref[...]

    h = A * (A - lt2)
    gsel = (jax.lax.broadcasted_iota(jnp.int32, (128, 4), 0) // D
            == jax.lax.broadcasted_iota(jnp.int32, (128, 4), 1)
            ).astype(jnp.float32)
    s4 = jax.lax.dot_general(
        h, gsel,
        dimension_numbers=(((2,), (0,)), ((), ())),
        preferred_element_type=jnp.float32)
    lsq = jnp.sum(lt2 * lt2, axis=2, keepdims=True) / 16.0
    d = s4 + lsq

    m = jnp.max(lab, axis=2, keepdims=True)
    iota_c = jax.lax.broadcasted_iota(jnp.int32, lab.shape, 2)
    is_max = lab == m
    first_idx = jnp.min(jnp.where(is_max, iota_c, C), axis=2, keepdims=True)
    last_idx = jnp.max(jnp.where(is_max, iota_c, -1), axis=2, keepdims=True)

    j4 = (4 * jax.lax.broadcasted_iota(jnp.int32, d.shape, 1)
          + jax.lax.broadcasted_iota(jnp.int32, d.shape, 2))
    valid = j4 < C

    d_correct = jnp.sum(
        jnp.where(j4 == first_idx, d, 0.0), axis=(1, 2))
    hardest = jnp.min(
        jnp.where(valid & (j4 != last_idx), d, jnp.float32(jnp.inf)),
        axis=(1, 2))
    loss = jnp.sum(jnp.maximum(d_correct - _ALPHA * hardest, 0.0))

    @pl.when(step == 0)
    def _init():
        out_ref[...] = jnp.zeros_like(out_ref)

    out_ref[...] = out_ref[...] + loss


def kernel(logits, labels, ans_emb, print_info):
    B, C = labels.shape
    D = logits.shape[1]
    nb = 8
    flat = ans_emb.reshape(B, C * D)
    lt2 = jnp.tile(2.0 * logits, (1, 4)).reshape(B, 1, 4 * D)
    body = functools.partial(_body, C=C, D=D)
    out = pl.pallas_call(
        body,
        grid=(B // nb,),
        in_specs=[
            pl.BlockSpec((nb, 1, 4 * D), lambda i: (i, 0, 0)),
            pl.BlockSpec((nb, 1, C), lambda i: (i, 0, 0)),
            pl.BlockSpec((nb, C * D), lambda i: (i, 0)),
        ],
        out_specs=pl.BlockSpec((1, 1), lambda i: (0, 0)),
        out_shape=jax.ShapeDtypeStruct((1, 1), jnp.float32),
        scratch_shapes=[pltpu.VMEM((nb, _R, 128), jnp.float32)],
    )(lt2, labels.reshape(B, 1, C), flat)
    return out[0, 0]


# speedup vs baseline: 8.7632x; 1.0505x over previous
import functools

import jax
import jax.numpy as jnp
from jax.experimental import pallas as pl
from jax.experimental.pallas import tpu as pltpu

_ALPHA = 0.5
_R = 784


def _body(lt2_ref, labels_ref, emb_ref, out_ref, a_scr, *, C, D):
    step = pl.program_id(0)
    nb = lt2_ref.shape[0]
    nfull = (C * D) // 128

    for r in range(nfull):
        a_scr[:, pl.ds(r, 1), :] = emb_ref[:, pl.ds(r * 128, 128)].reshape(
            nb, 1, 128)
    tail = emb_ref[:, pl.ds(nfull * 128, C * D - nfull * 128)]
    a_scr[:, pl.ds(nfull, 1), :] = jnp.concatenate(
        [tail, jnp.zeros((nb, 96), jnp.float32)], axis=1).reshape(nb, 1, 128)

    @pl.when(step == 0)
    def _zero_tail():
        a_scr[:, pl.ds(nfull + 1, _R - nfull - 1), :] = jnp.zeros(
            (nb, _R - nfull - 1, 128), jnp.float32)

    A = a_scr[...]
    lt2 = lt2_ref[...]
    lab = labels_ref[...]

    h = A * (A - lt2)
    gsel = (jax.lax.broadcasted_iota(jnp.int32, (128, 4), 0) // D
            == jax.lax.broadcasted_iota(jnp.int32, (128, 4), 1)
            ).astype(jnp.float32)
    s4 = jax.lax.dot_general(
        h, gsel,
        dimension_numbers=(((2,), (0,)), ((), ())),
        preferred_element_type=jnp.float32)
    lsq = jnp.sum(lt2 * lt2, axis=2, keepdims=True) / 16.0
    d = s4 + lsq

    m = jnp.max(lab, axis=2, keepdims=True)
    iota_c = jax.lax.broadcasted_iota(jnp.int32, lab.shape, 2)
    is_max = lab == m
    first_idx = jnp.min(jnp.where(is_max, iota_c, C), axis=2, keepdims=True)
    last_idx = jnp.max(jnp.where(is_max, iota_c, -1), axis=2, keepdims=True)

    j4 = (4 * jax.lax.broadcasted_iota(jnp.int32, d.shape, 1)
          + jax.lax.broadcasted_iota(jnp.int32, d.shape, 2))
    valid = j4 < C

    d_correct = jnp.sum(
        jnp.where(j4 == first_idx, d, 0.0), axis=(1, 2))
    hardest = jnp.min(
        jnp.where(valid & (j4 != last_idx), d, jnp.float32(jnp.inf)),
        axis=(1, 2))
    loss = jnp.sum(jnp.maximum(d_correct - _ALPHA * hardest, 0.0))

    @pl.when(step == 0)
    def _init():
        out_ref[...] = jnp.zeros_like(out_ref)

    out_ref[...] = out_ref[...] + loss


def kernel(logits, labels, ans_emb, print_info):
    B, C = labels.shape
    D = logits.shape[1]
    nb = 16
    flat = ans_emb.reshape(B, C * D)
    lt2 = jnp.tile(2.0 * logits, (1, 4)).reshape(B, 1, 4 * D)
    body = functools.partial(_body, C=C, D=D)
    out = pl.pallas_call(
        body,
        grid=(B // nb,),
        in_specs=[
            pl.BlockSpec((nb, 1, 4 * D), lambda i: (i, 0, 0)),
            pl.BlockSpec((nb, 1, C), lambda i: (i, 0, 0)),
            pl.BlockSpec((nb, C * D), lambda i: (i, 0)),
        ],
        out_specs=pl.BlockSpec((1, 1), lambda i: (0, 0)),
        out_shape=jax.ShapeDtypeStruct((1, 1), jnp.float32),
        scratch_shapes=[pltpu.VMEM((nb, _R, 128), jnp.float32)],
    )(lt2, labels.reshape(B, 1, C), flat)
    return out[0, 0]
